# baseline (speedup 1.0000x reference)
import functools

import jax
import jax.numpy as jnp
import numpy as np
from jax.experimental import pallas as pl
from jax.experimental.pallas import tpu as pltpu

_R = 10
_MAXR = 2 * _R - 1
_TINY = np.float32(np.finfo(np.float32).tiny)


def _threefry_xor_bits(k0, k1, cnt):
    ks2 = k0 ^ k1 ^ np.uint32(0x1BD11BDA)
    ks = (k0, k1, ks2)

    def rotl(v, d):
        return (v << np.uint32(d)) | (v >> np.uint32(32 - d))

    rots = ((13, 15, 26, 6), (17, 29, 16, 24))
    x1 = cnt + k1
    x0 = x1 + k0
    x1 = rotl(x1, 13)
    x1 = x1 ^ x0
    first = True
    for i in range(5):
        for r in rots[i % 2]:
            if first:
                first = False
                continue
            x0 = x0 + x1
            x1 = rotl(x1, r)
            x1 = x1 ^ x0
        x0 = x0 + ks[(i + 1) % 3]
        x1 = x1 + ks[(i + 2) % 3] + np.uint32(i + 1)
    return x0 ^ x1


def _neg_gumbel_from_bits(bits):
    f = jax.lax.bitcast_convert_type(
        (bits >> np.uint32(9)) | np.uint32(0x3F800000), jnp.float32) - 1.0
    u = jnp.maximum(f, _TINY)
    return jnp.log(-jnp.log(u))


def _sampler_block(perm_ref, x_hbm, w_ref, rad_ref, u_ref, row_ref, keys_ref,
                   o_hbm, xbuf, obuf, sgn_ref, in_sem, out_sem,
                   *, rblk, dim, nblk):
    i = pl.program_id(0)

    def in_copy(blk, slot):
        def issue(r, c):
            src_row = perm_ref[blk * rblk + r]
            return pltpu.make_async_copy(
                x_hbm.at[pl.ds(src_row, 1), :],
                xbuf.at[slot, pl.ds(r, 1), :],
                in_sem)
        return issue

    def out_copy(blk, slot):
        def issue(r, c):
            dst_row = perm_ref[blk * rblk + r]
            return pltpu.make_async_copy(
                obuf.at[slot, pl.ds(r, 1), :],
                o_hbm.at[pl.ds(dst_row, 1), :],
                out_sem)
        return issue

    def start_in(blk, slot):
        def body(r, c):
            in_copy(blk, slot)(r, None).start()
            return c
        jax.lax.fori_loop(0, rblk, body, 0)

    def wait_in(blk, slot):
        def body(r, c):
            in_copy(blk, slot)(r, None).wait()
            return c
        jax.lax.fori_loop(0, rblk, body, 0)

    def start_out(blk, slot):
        def body(r, c):
            out_copy(blk, slot)(r, None).start()
            return c
        jax.lax.fori_loop(0, rblk, body, 0)

    def wait_out(blk, slot):
        def body(r, c):
            out_copy(blk, slot)(r, None).wait()
            return c
        jax.lax.fori_loop(0, rblk, body, 0)

    @pl.when(i == 0)
    def _():
        start_in(0, 0)

    slot = jax.lax.rem(i, 2)
    wait_in(i, slot)

    @pl.when(i + 1 < nblk)
    def _():
        start_in(i + 1, 1 - slot)

    x0 = jnp.where(slot == 0, xbuf[0], xbuf[1])
    w = w_ref[...]
    wh = w * np.float32(0.5)

    col = jax.lax.broadcasted_iota(jnp.int32, (rblk, dim), 1)
    flat = row_ref[...] * np.uint32(dim) + \
        jax.lax.broadcasted_iota(jnp.uint32, (rblk, dim), 1)
    whb = jax.lax.bitcast_convert_type(jnp.broadcast_to(wh, (rblk, dim)), jnp.uint32)

    sgn_ref[...] = jnp.where(x0 != 0.0, np.uint32(0x80000000), np.uint32(0))

    s0 = jax.lax.bitcast_convert_type(whb ^ sgn_ref[...], jnp.float32)
    m0 = jnp.max(s0, axis=-1, keepdims=True)
    log_zx = jnp.log(jnp.sum(jnp.exp(s0 - m0), axis=-1, keepdims=True)) + m0
    score_x = jnp.sum(x0 * w, axis=-1, keepdims=True)
    rad = rad_ref[...]
    t_max = jnp.max(rad)

    def step(t, carry):
        sg = sgn_ref[...]
        s = jax.lax.bitcast_convert_type(whb ^ sg, jnp.float32)
        bits = _threefry_xor_bits(keys_ref[t, 0], keys_ref[t, 1], flat)
        v = s - _neg_gumbel_from_bits(bits)
        m = jnp.max(v, axis=-1, keepdims=True)
        idx = jnp.min(jnp.where(v == m, col, np.int32(dim)), axis=-1, keepdims=True)
        idx = jnp.where(t < rad, idx, np.int32(dim))
        flip = jnp.where(col == idx, np.uint32(0x80000000), np.uint32(0))
        sgn_ref[...] = sg ^ flip
        return carry

    jax.lax.fori_loop(0, t_max, step, 0, unroll=False)

    sgn = sgn_ref[...]
    y = (sgn >> np.uint32(31)).astype(jnp.float32)
    s_y = jax.lax.bitcast_convert_type(whb ^ sgn, jnp.float32)
    my = jnp.max(s_y, axis=-1, keepdims=True)
    lse_y = jnp.log(jnp.sum(jnp.exp(s_y - my), axis=-1, keepdims=True)) + my
    score_y = jnp.sum(y * w, axis=-1, keepdims=True)
    log_tilde = -jnp.sum(w * (y - x0), axis=-1, keepdims=True)
    log_acc = jnp.minimum((score_y - score_x) + log_tilde + (log_zx - lse_y), 0.0)
    acc = jnp.exp(log_acc) >= u_ref[...]
    res = jnp.where(acc, y, x0)

    @pl.when(slot == 0)
    def _():
        obuf[0] = res

    @pl.when(slot == 1)
    def _():
        obuf[1] = res

    start_out(i, slot)
    wait_out(i, slot)


@jax.jit
def kernel(x, W):
    bsize, dim = x.shape
    key = jax.random.key(42)
    k_rad, k_loop, k_acc = jax.random.split(key, 3)
    radius = jax.random.randint(k_rad, (bsize, 1), 1, 2 * _R)
    u_acc = jax.random.uniform(k_acc, (bsize,), dtype=jnp.float32)
    step_keys = jnp.stack(
        [jax.random.key_data(jax.random.fold_in(k_loop, t)) for t in range(_MAXR)])

    rblk = 128
    nblk = bsize // rblk

    rad_flat = radius[:, 0]
    perm = jnp.argsort(rad_flat)
    half = nblk // 2
    order = np.empty((nblk,), np.int32)
    order[0::2] = np.arange(half)
    order[1::2] = np.arange(nblk - 1, half - 1, -1)
    perm = perm.reshape(nblk, rblk)[order].reshape(-1)

    radp = rad_flat[perm][:, None]
    up = u_acc[perm][:, None]
    rowp = perm.astype(jnp.uint32)[:, None]

    body = functools.partial(_sampler_block, rblk=rblk, dim=dim, nblk=nblk)
    out = pl.pallas_call(
        body,
        grid_spec=pltpu.PrefetchScalarGridSpec(
            num_scalar_prefetch=1,
            grid=(nblk,),
            in_specs=[
                pl.BlockSpec(memory_space=pl.ANY),
                pl.BlockSpec((1, dim), lambda i, p: (0, 0)),
                pl.BlockSpec((rblk, 1), lambda i, p: (i, 0)),
                pl.BlockSpec((rblk, 1), lambda i, p: (i, 0)),
                pl.BlockSpec((rblk, 1), lambda i, p: (i, 0)),
                pl.BlockSpec(memory_space=pltpu.SMEM),
            ],
            out_specs=pl.BlockSpec(memory_space=pl.ANY),
            scratch_shapes=[
                pltpu.VMEM((2, rblk, dim), jnp.float32),
                pltpu.VMEM((2, rblk, dim), jnp.float32),
                pltpu.VMEM((rblk, dim), jnp.uint32),
                pltpu.SemaphoreType.DMA,
                pltpu.SemaphoreType.DMA,
            ],
        ),
        out_shape=jax.ShapeDtypeStruct((bsize, dim), jnp.float32),
        compiler_params=pltpu.CompilerParams(
            dimension_semantics=("arbitrary",),
        ),
    )(perm.astype(jnp.int32), x, W.reshape(1, dim), radp, up, rowp, step_keys)
    return out


# speedup vs baseline: 1.0036x; 1.0036x over previous
import functools

import jax
import jax.numpy as jnp
import numpy as np
from jax.experimental import pallas as pl
from jax.experimental.pallas import tpu as pltpu

_R = 10
_MAXR = 2 * _R - 1
_TINY = np.float32(np.finfo(np.float32).tiny)


def _threefry_xor_bits(k0, k1, cnt):
    ks2 = k0 ^ k1 ^ np.uint32(0x1BD11BDA)
    ks = (k0, k1, ks2)

    def rotl(v, d):
        return (v << np.uint32(d)) | (v >> np.uint32(32 - d))

    rots = ((13, 15, 26, 6), (17, 29, 16, 24))
    x1 = cnt + k1
    x0 = x1 + k0
    x1 = rotl(x1, 13)
    x1 = x1 ^ x0
    first = True
    for i in range(5):
        for r in rots[i % 2]:
            if first:
                first = False
                continue
            x0 = x0 + x1
            x1 = rotl(x1, r)
            x1 = x1 ^ x0
        x0 = x0 + ks[(i + 1) % 3]
        x1 = x1 + ks[(i + 2) % 3] + np.uint32(i + 1)
    return x0 ^ x1


def _gumbel_from_bits(bits):
    f = jax.lax.bitcast_convert_type(
        (bits >> np.uint32(9)) | np.uint32(0x3F800000), jnp.float32) - 1.0
    u = jnp.maximum(f, _TINY)
    return -jnp.log(-jnp.log(u))


def _sampler_block(x_ref, w_ref, rad_ref, u_ref, row_ref, keys_ref, o_ref, *, rblk, dim):
    x0 = x_ref[...]
    w = w_ref[...]
    wh = w * np.float32(0.5)

    def signed_logits(xb):
        return wh - xb * w

    col = jax.lax.broadcasted_iota(jnp.int32, (rblk, dim), 1)
    flat = row_ref[...] * np.uint32(dim) + \
        jax.lax.broadcasted_iota(jnp.uint32, (rblk, dim), 1)

    s0 = signed_logits(x0)
    m0 = jnp.max(s0, axis=-1, keepdims=True)
    log_zx = jnp.log(jnp.sum(jnp.exp(s0 - m0), axis=-1, keepdims=True)) + m0
    score_x = jnp.sum(x0 * w, axis=-1, keepdims=True)
    rad = rad_ref[...]
    t_max = jnp.max(rad)

    o_ref[...] = x0

    def step(t, carry):
        xc = o_ref[...]
        s = signed_logits(xc)
        bits = _threefry_xor_bits(keys_ref[t, 0], keys_ref[t, 1], flat)
        v = _gumbel_from_bits(bits) + s
        m = jnp.max(v, axis=-1, keepdims=True)
        idx = jnp.min(jnp.where(v == m, col, np.int32(dim)), axis=-1, keepdims=True)
        idx = jnp.where(t < rad, idx, np.int32(dim))
        mask = col == idx
        o_ref[...] = jnp.where(mask, 1.0 - xc, xc)
        return carry

    jax.lax.fori_loop(0, t_max, step, 0, unroll=False)

    y = o_ref[...]
    s_y = signed_logits(y)
    my = jnp.max(s_y, axis=-1, keepdims=True)
    lse_y = jnp.log(jnp.sum(jnp.exp(s_y - my), axis=-1, keepdims=True)) + my
    score_y = jnp.sum(y * w, axis=-1, keepdims=True)
    log_tilde = -jnp.sum(w * (y - x0), axis=-1, keepdims=True)
    log_acc = jnp.minimum((score_y - score_x) + log_tilde + (log_zx - lse_y), 0.0)
    acc = jnp.exp(log_acc) >= u_ref[...]
    o_ref[...] = jnp.where(acc, y, x0)


@jax.jit
def kernel(x, W):
    bsize, dim = x.shape
    key = jax.random.key(42)
    k_rad, k_loop, k_acc = jax.random.split(key, 3)
    radius = jax.random.randint(k_rad, (bsize, 1), 1, 2 * _R)
    u_acc = jax.random.uniform(k_acc, (bsize,), dtype=jnp.float32)
    step_keys = jnp.stack(
        [jax.random.key_data(jax.random.fold_in(k_loop, t)) for t in range(_MAXR)])

    rblk = 128
    nblk = bsize // rblk

    rad_flat = radius[:, 0]
    perm = jnp.argsort(rad_flat)
    half = nblk // 2
    order = np.empty((nblk,), np.int32)
    order[0::2] = np.arange(half)
    order[1::2] = np.arange(nblk - 1, half - 1, -1)
    perm = perm.reshape(nblk, rblk)[order].reshape(-1)
    inv = jnp.argsort(perm)

    xp = x[perm]
    radp = rad_flat[perm][:, None]
    up = u_acc[perm][:, None]
    rowp = perm.astype(jnp.uint32)[:, None]

    body = functools.partial(_sampler_block, rblk=rblk, dim=dim)
    out_p = pl.pallas_call(
        body,
        grid=(nblk,),
        in_specs=[
            pl.BlockSpec((rblk, dim), lambda i: (i, 0)),
            pl.BlockSpec((1, dim), lambda i: (0, 0)),
            pl.BlockSpec((rblk, 1), lambda i: (i, 0)),
            pl.BlockSpec((rblk, 1), lambda i: (i, 0)),
            pl.BlockSpec((rblk, 1), lambda i: (i, 0)),
            pl.BlockSpec(memory_space=pltpu.SMEM),
        ],
        out_specs=pl.BlockSpec((rblk, dim), lambda i: (i, 0)),
        out_shape=jax.ShapeDtypeStruct((bsize, dim), jnp.float32),
        compiler_params=pltpu.CompilerParams(
            dimension_semantics=("parallel",),
        ),
    )(xp, W.reshape(1, dim), radp, up, rowp, step_keys)
    return out_p[inv]
